# baseline (speedup 1.0000x reference)
import functools

import jax
import jax.numpy as jnp
from jax import lax
from jax.experimental import pallas as pl
from jax.experimental.pallas import tpu as pltpu
from jax.experimental.pallas import tpu_sc as plsc

NC = 2
NS = 16
NW = NC * NS
CH = 128
SCH = 64


def _ru(n, m):
    return (n + m - 1) // m * m


def _padw(w, r, c):
    return jnp.pad(w, ((0, r - w.shape[0]), (0, c - w.shape[1])))


def _linear_body(x_ref, w_ref, b_ref, o_ref):
    o_ref[...] = (
        jnp.dot(x_ref[...], w_ref[...], preferred_element_type=jnp.float32)
        + b_ref[...]
    )


def _node_linear(xp, w, b, blk=512):
    n, k = xp.shape
    dp = w.shape[1]
    return pl.pallas_call(
        _linear_body,
        grid=(n // blk,),
        in_specs=[
            pl.BlockSpec((blk, k), lambda i: (i, 0)),
            pl.BlockSpec((k, dp), lambda i: (0, 0)),
            pl.BlockSpec((1, dp), lambda i: (0, 0)),
        ],
        out_specs=pl.BlockSpec((blk, dp), lambda i: (i, 0)),
        out_shape=jax.ShapeDtypeStruct((n, dp), jnp.float32),
    )(xp, w, b)


def _attr_body(ea_ref, aw1_ref, ab1_ref, aw2_ref, ab2_ref, mw1e_ref,
               ean_ref, g_ref):
    h = jnp.maximum(
        jnp.dot(ea_ref[...], aw1_ref[...], preferred_element_type=jnp.float32)
        + ab1_ref[...], 0.0)
    ean = (jnp.dot(h, aw2_ref[...], preferred_element_type=jnp.float32)
           + ab2_ref[...])
    ean_ref[...] = ean
    g_ref[...] = jnp.dot(ean, mw1e_ref[...], preferred_element_type=jnp.float32)


def _attr_step(ea, kaw1, kab1, kaw2, kab2, kmw1e, blk=256):
    ep8, w16 = ea.shape
    wrow = kmw1e.shape[1]
    return pl.pallas_call(
        _attr_body,
        grid=(ep8 // blk,),
        in_specs=[
            pl.BlockSpec((blk, w16), lambda i: (i, 0)),
            pl.BlockSpec((w16, w16), lambda i: (0, 0)),
            pl.BlockSpec((1, w16), lambda i: (0, 0)),
            pl.BlockSpec((w16, w16), lambda i: (0, 0)),
            pl.BlockSpec((1, w16), lambda i: (0, 0)),
            pl.BlockSpec((w16, wrow), lambda i: (0, 0)),
        ],
        out_specs=[
            pl.BlockSpec((blk, w16), lambda i: (i, 0)),
            pl.BlockSpec((blk, wrow), lambda i: (i, 0)),
        ],
        out_shape=[
            jax.ShapeDtypeStruct((ep8, w16), jnp.float32),
            jax.ShapeDtypeStruct((ep8, wrow), jnp.float32),
        ],
    )(ea, kaw1, kab1, kaw2, kab2, kmw1e)


def _post_body(p_ref, c_ref, h_ref, ab1_ref, aw2_ref, ab2_ref, mw1e_ref,
               mw2_ref, mb2_ref, wn_ref, bn_ref, o_ref, *, last):
    gs = (jnp.dot(jnp.maximum(ab1_ref[...], 0.0), aw2_ref[...],
                  preferred_element_type=jnp.float32) + ab2_ref[...])
    gself = jnp.dot(gs, mw1e_ref[...], preferred_element_type=jnp.float32)
    acc = p_ref[0] + p_ref[1] + jnp.maximum(h_ref[...] + gself, 0.0)
    cnt = c_ref[0, :, 0:1] + c_ref[1, :, 0:1] + 1.0
    m = acc / cnt
    out = (jnp.dot(m, mw2_ref[...], preferred_element_type=jnp.float32)
           + mb2_ref[...])
    if last:
        o_ref[...] = out
    else:
        out = jnp.maximum(out, 0.0)
        o_ref[...] = (jnp.dot(out, wn_ref[...],
                              preferred_element_type=jnp.float32)
                      + bn_ref[...])


def _post_step(p, cntp, h, ab1, aw2, ab2, mw1e, mw2, mb2, wn, bn, last,
               blk=512):
    npad, dp = h.shape
    de = aw2.shape[0]
    dn = wn.shape[1]
    return pl.pallas_call(
        functools.partial(_post_body, last=last),
        grid=(npad // blk,),
        in_specs=[
            pl.BlockSpec((2, blk, dp), lambda i: (0, i, 0)),
            pl.BlockSpec((2, blk, 128), lambda i: (0, i, 0)),
            pl.BlockSpec((blk, dp), lambda i: (i, 0)),
            pl.BlockSpec((1, de), lambda i: (0, 0)),
            pl.BlockSpec((de, de), lambda i: (0, 0)),
            pl.BlockSpec((1, de), lambda i: (0, 0)),
            pl.BlockSpec((de, dp), lambda i: (0, 0)),
            pl.BlockSpec((dp, dp), lambda i: (0, 0)),
            pl.BlockSpec((1, dp), lambda i: (0, 0)),
            pl.BlockSpec((dp, dn), lambda i: (0, 0)),
            pl.BlockSpec((1, dn), lambda i: (0, 0)),
        ],
        out_specs=pl.BlockSpec((blk, dn), lambda i: (i, 0)),
        out_shape=jax.ShapeDtypeStruct((npad, dn), jnp.float32),
    )(p, cntp, h, ab1, aw2, ab2, mw1e, mw2, mb2, wn, bn)


def _mesh():
    return plsc.VectorSubcoreMesh(
        core_axis_name="c", subcore_axis_name="s", num_cores=NC,
        num_subcores=NS)


def _sc_scatter(h, g, src3, dst3, dpt):
    npad, dp = h.shape
    nw, t, sch = dst3.shape
    wrow = g.shape[1]
    gpc = sch // 8
    rt = npad // NS

    def body(h_hbm, g_hbm, src_hbm, dst_hbm, acc_out,
             srcv, dstv, r0, r1, g0, g1, accsh,
             ga0, ga1, gb0, gb1, ss0, ss1):
        c = lax.axis_index("c")
        s = lax.axis_index("s")
        w = c * NS + s
        rows = (r0, r1)
        gvs = (g0, g1)
        ga = (ga0, ga1)
        gb = (gb0, gb1)
        ss = (ss0, ss1)

        pltpu.sync_copy(src_hbm.at[w], srcv)
        pltpu.sync_copy(dst_hbm.at[w], dstv)

        z = jnp.zeros((16,), jnp.float32)
        for r in range(16):
            for q in range(dp // 16):
                r0[r, q * 16:(q + 1) * 16] = z

        def zl(j, carry):
            pltpu.sync_copy(r0.at[pl.ds(0, 16)],
                            accsh.at[pl.ds(s * rt + j * 16, 16)])
            return carry

        lax.fori_loop(0, rt // 16, zl, 0)
        plsc.subcore_barrier()

        def issue_gather(kk, cell):
            return pltpu.async_copy(h_hbm.at[srcv.at[pl.ds(kk * sch, sch)]],
                                    rows[cell], ga[cell])

        def wait_gather(cell):
            pltpu.make_async_copy(h_hbm.at[srcv.at[pl.ds(0, sch)]],
                                  rows[cell], ga[cell]).wait()

        def issue_g(kk, cell):
            base = (w * t + kk) * gpc
            return pltpu.async_copy(g_hbm.at[pl.ds(base, gpc)], gvs[cell],
                                    gb[cell])

        def wait_g(cell):
            pltpu.make_async_copy(g_hbm.at[pl.ds(0, gpc)], gvs[cell],
                                  gb[cell]).wait()

        def issue_scatter(k, cell):
            return pltpu.async_copy(rows[cell], accsh.at[dstv.at[k]],
                                    ss[cell], add=True)

        def wait_scatter(cell):
            pltpu.make_async_copy(rows[cell], accsh.at[dstv.at[0]],
                                  ss[cell]).wait()

        def ew(cell):
            rr, gg = rows[cell], gvs[cell]

            def erow(j, carry2):
                for u in range(8):
                    r = j * 8 + u
                    for q in range(dpt // 16):
                        sl = pl.ds(q * 16, 16)
                        slg = pl.ds(u * dpt + q * 16, 16)
                        rr[r, sl] = jnp.maximum(rr[r, sl] + gg[j, slg], 0.0)
                return carry2

            lax.fori_loop(0, gpc, erow, 0)

        issue_gather(0, 0)
        issue_g(0, 0)
        wait_gather(0)
        wait_g(0)
        issue_gather(1, 1)
        issue_g(1, 1)
        ew(0)
        issue_scatter(0, 0)
        wait_gather(1)
        wait_g(1)
        wait_scatter(0)
        issue_gather(2, 0)
        issue_g(2, 0)
        ew(1)
        issue_scatter(1, 1)

        def pair(j, carry):
            for u in range(2):
                k = 2 + 2 * j + u
                b = u
                wait_gather(b)
                wait_g(b)
                kk = jnp.minimum(k + 1, t - 1)
                wait_scatter(1 - b)
                issue_gather(kk, 1 - b)
                issue_g(kk, 1 - b)
                ew(b)
                issue_scatter(k, b)
            return carry

        lax.fori_loop(0, (t - 2) // 2, pair, 0)

        wait_gather(0)
        wait_g(0)
        wait_scatter(1)
        plsc.subcore_barrier()

        def co(j, carry):
            off = s * rt + j * sch
            pltpu.sync_copy(accsh.at[pl.ds(off, sch)], r0)
            pltpu.sync_copy(r0, acc_out.at[c, pl.ds(off, sch)])
            return carry

        lax.fori_loop(0, rt // sch, co, 0)

    fn = pl.kernel(
        body,
        out_type=[jax.ShapeDtypeStruct((NC, npad, dp), jnp.float32)],
        mesh=_mesh(),
        scratch_types=[
            pltpu.VMEM((t * sch,), jnp.int32),
            pltpu.VMEM((t, sch), jnp.int32),
            pltpu.VMEM((sch, dp), jnp.float32),
            pltpu.VMEM((sch, dp), jnp.float32),
            pltpu.VMEM((gpc, wrow), jnp.float32),
            pltpu.VMEM((gpc, wrow), jnp.float32),
            pltpu.VMEM_SHARED((npad, dp), jnp.float32),
            pltpu.SemaphoreType.DMA,
            pltpu.SemaphoreType.DMA,
            pltpu.SemaphoreType.DMA,
            pltpu.SemaphoreType.DMA,
            pltpu.SemaphoreType.DMA,
            pltpu.SemaphoreType.DMA,
        ])
    (out,) = fn(h, g, src3, dst3)
    return out


def _sc_degree(dstp, npad):
    ep = dstp.shape[0]
    t = ep // (NC * NS * CH)
    rt = npad // NS
    dp = 128

    def body(dst_hbm, cnt_out, dstv, onesv, cntsh):
        c = lax.axis_index("c")
        s = lax.axis_index("s")
        w = c * NS + s

        z = jnp.zeros((16,), jnp.float32)
        for r in range(16):
            for q in range(dp // 16):
                onesv[r, q * 16:(q + 1) * 16] = z

        def zl(j, carry):
            pltpu.sync_copy(onesv.at[pl.ds(0, 16)],
                            cntsh.at[pl.ds(s * rt + j * 16, 16)])
            return carry

        lax.fori_loop(0, rt // 16, zl, 0)

        one = jnp.ones((16,), jnp.float32)

        def ol(r, carry):
            for q in range(dp // 16):
                onesv[r, q * 16:(q + 1) * 16] = one
            return carry

        lax.fori_loop(0, CH, ol, 0)
        plsc.subcore_barrier()

        def chunk(k, carry):
            base = (w * t + k) * CH
            pltpu.sync_copy(dst_hbm.at[pl.ds(base, CH)], dstv)
            pltpu.sync_copy(onesv, cntsh.at[dstv], add=True)
            return carry

        lax.fori_loop(0, t, chunk, 0)
        plsc.subcore_barrier()

        def co(j, carry):
            off = s * rt + j * CH
            pltpu.sync_copy(cntsh.at[pl.ds(off, CH)], onesv)
            pltpu.sync_copy(onesv, cnt_out.at[c, pl.ds(off, CH)])
            return carry

        lax.fori_loop(0, rt // CH, co, 0)

    fn = pl.kernel(
        body,
        out_type=[jax.ShapeDtypeStruct((NC, npad, dp), jnp.float32)],
        mesh=_mesh(),
        scratch_types=[
            pltpu.VMEM((CH,), jnp.int32),
            pltpu.VMEM((CH, dp), jnp.float32),
            pltpu.VMEM_SHARED((npad, dp), jnp.float32),
        ])
    (out,) = fn(dstp)
    return out


def kernel(x, edge_index, edge_attr, params):
    n, d = x.shape
    e = edge_index.shape[1]
    de = edge_attr.shape[1]
    nlayers = len(params)

    npad = _ru(n + 1, 2048)
    ep = _ru(e, 2 * NW * CH)

    pad_dst = n + jnp.arange(ep - e, dtype=edge_index.dtype) % (npad - n)
    src = jnp.concatenate(
        [edge_index[0], jnp.zeros((ep - e,), edge_index.dtype)])
    dst = jnp.concatenate([edge_index[1], pad_dst])
    t = ep // (NW * SCH)
    src3 = src.reshape(NW, t * SCH)
    dst3 = dst.reshape(NW, t, SCH)
    xp = jnp.pad(x, ((0, npad - n), (0, 0)))
    ea = jnp.pad(edge_attr, ((0, ep - e), (0, 0))).reshape(ep // 8, 8 * de)

    dps = [_ru(p["mW1"].shape[1], 128) for p in params]
    dpts = [_ru(p["mW1"].shape[1], 16) for p in params]
    douts = [p["mW1"].shape[1] for p in params]
    dins = [p["mW1"].shape[0] - de for p in params]
    i8 = jnp.eye(8, dtype=jnp.float32)

    lp0 = params[0]
    h = _node_linear(
        xp,
        _padw(lp0["mW1"][: dins[0]], d, dps[0]),
        _padw(lp0["mb1"][None, :], 1, dps[0]),
    )

    cntp = _sc_degree(dst, npad)
    for i, lp in enumerate(params):
        dp = dps[i]
        dpt = dpts[i]
        mw1e = _padw(lp["mW1"][dins[i]:], de, dp)
        ab1 = lp["ab1"][None, :]
        ab2 = lp["ab2"][None, :]
        ea, g = _attr_step(
            ea,
            jnp.kron(i8, lp["aW1"]), jnp.tile(ab1, (1, 8)),
            jnp.kron(i8, lp["aW2"]), jnp.tile(ab2, (1, 8)),
            jnp.kron(i8, _padw(lp["mW1"][dins[i]:], de, dpt)))
        p = _sc_scatter(h, g, src3, dst3, dpt)
        last = i == nlayers - 1
        if last:
            wn = jnp.zeros((dp, dp), jnp.float32)
            bn = jnp.zeros((1, dp), jnp.float32)
        else:
            nxt = params[i + 1]
            wn = _padw(nxt["mW1"][: dins[i + 1]], dp, dps[i + 1])
            bn = _padw(nxt["mb1"][None, :], 1, dps[i + 1])
        h = _post_step(
            p, cntp, h, ab1, lp["aW2"], ab2, mw1e,
            _padw(lp["mW2"], dp, dp), _padw(lp["mb2"][None, :], 1, dp),
            wn, bn, last)

    return h[:n, : douts[-1]]


# speedup vs baseline: 1.0286x; 1.0286x over previous
import functools

import jax
import jax.numpy as jnp
from jax import lax
from jax.experimental import pallas as pl
from jax.experimental.pallas import tpu as pltpu
from jax.experimental.pallas import tpu_sc as plsc

NC = 2
NS = 16
NW = NC * NS
CH = 128
SCH = 64


def _ru(n, m):
    return (n + m - 1) // m * m


def _padw(w, r, c):
    return jnp.pad(w, ((0, r - w.shape[0]), (0, c - w.shape[1])))


def _linear_body(x_ref, w_ref, b_ref, o_ref):
    o_ref[...] = (
        jnp.dot(x_ref[...], w_ref[...], preferred_element_type=jnp.float32)
        + b_ref[...]
    )


def _node_linear(xp, w, b, blk=512):
    n, k = xp.shape
    dp = w.shape[1]
    return pl.pallas_call(
        _linear_body,
        grid=(n // blk,),
        in_specs=[
            pl.BlockSpec((blk, k), lambda i: (i, 0)),
            pl.BlockSpec((k, dp), lambda i: (0, 0)),
            pl.BlockSpec((1, dp), lambda i: (0, 0)),
        ],
        out_specs=pl.BlockSpec((blk, dp), lambda i: (i, 0)),
        out_shape=jax.ShapeDtypeStruct((n, dp), jnp.float32),
    )(xp, w, b)


def _attr_body(ea_ref, aw1_ref, ab1_ref, aw2_ref, ab2_ref, mw1e_ref,
               ean_ref, g_ref):
    h = jnp.maximum(
        jnp.dot(ea_ref[...], aw1_ref[...], preferred_element_type=jnp.float32)
        + ab1_ref[...], 0.0)
    ean = (jnp.dot(h, aw2_ref[...], preferred_element_type=jnp.float32)
           + ab2_ref[...])
    ean_ref[...] = ean
    g_ref[...] = jnp.dot(ean, mw1e_ref[...], preferred_element_type=jnp.float32)


def _attr_step(ea, kaw1, kab1, kaw2, kab2, kmw1e, blk=256):
    ep8, w16 = ea.shape
    wrow = kmw1e.shape[1]
    return pl.pallas_call(
        _attr_body,
        grid=(ep8 // blk,),
        in_specs=[
            pl.BlockSpec((blk, w16), lambda i: (i, 0)),
            pl.BlockSpec((w16, w16), lambda i: (0, 0)),
            pl.BlockSpec((1, w16), lambda i: (0, 0)),
            pl.BlockSpec((w16, w16), lambda i: (0, 0)),
            pl.BlockSpec((1, w16), lambda i: (0, 0)),
            pl.BlockSpec((w16, wrow), lambda i: (0, 0)),
        ],
        out_specs=[
            pl.BlockSpec((blk, w16), lambda i: (i, 0)),
            pl.BlockSpec((blk, wrow), lambda i: (i, 0)),
        ],
        out_shape=[
            jax.ShapeDtypeStruct((ep8, w16), jnp.float32),
            jax.ShapeDtypeStruct((ep8, wrow), jnp.float32),
        ],
    )(ea, kaw1, kab1, kaw2, kab2, kmw1e)


def _post_body(p_ref, c_ref, h_ref, ab1_ref, aw2_ref, ab2_ref, mw1e_ref,
               mw2_ref, mb2_ref, wn_ref, bn_ref, o_ref, *, last):
    gs = (jnp.dot(jnp.maximum(ab1_ref[...], 0.0), aw2_ref[...],
                  preferred_element_type=jnp.float32) + ab2_ref[...])
    gself = jnp.dot(gs, mw1e_ref[...], preferred_element_type=jnp.float32)
    acc = p_ref[0] + p_ref[1] + jnp.maximum(h_ref[...] + gself, 0.0)
    cnt = c_ref[0, :, 0:1] + c_ref[1, :, 0:1] + 1.0
    m = acc / cnt
    out = (jnp.dot(m, mw2_ref[...], preferred_element_type=jnp.float32)
           + mb2_ref[...])
    if last:
        o_ref[...] = out
    else:
        out = jnp.maximum(out, 0.0)
        o_ref[...] = (jnp.dot(out, wn_ref[...],
                              preferred_element_type=jnp.float32)
                      + bn_ref[...])


def _post_step(p, cntp, h, ab1, aw2, ab2, mw1e, mw2, mb2, wn, bn, last,
               blk=512):
    npad, dp = h.shape
    de = aw2.shape[0]
    dn = wn.shape[1]
    return pl.pallas_call(
        functools.partial(_post_body, last=last),
        grid=(npad // blk,),
        in_specs=[
            pl.BlockSpec((2, blk, dp), lambda i: (0, i, 0)),
            pl.BlockSpec((2, blk, 128), lambda i: (0, i, 0)),
            pl.BlockSpec((blk, dp), lambda i: (i, 0)),
            pl.BlockSpec((1, de), lambda i: (0, 0)),
            pl.BlockSpec((de, de), lambda i: (0, 0)),
            pl.BlockSpec((1, de), lambda i: (0, 0)),
            pl.BlockSpec((de, dp), lambda i: (0, 0)),
            pl.BlockSpec((dp, dp), lambda i: (0, 0)),
            pl.BlockSpec((1, dp), lambda i: (0, 0)),
            pl.BlockSpec((dp, dn), lambda i: (0, 0)),
            pl.BlockSpec((1, dn), lambda i: (0, 0)),
        ],
        out_specs=pl.BlockSpec((blk, dn), lambda i: (i, 0)),
        out_shape=jax.ShapeDtypeStruct((npad, dn), jnp.float32),
    )(p, cntp, h, ab1, aw2, ab2, mw1e, mw2, mb2, wn, bn)


def _mesh():
    return plsc.VectorSubcoreMesh(
        core_axis_name="c", subcore_axis_name="s", num_cores=NC,
        num_subcores=NS)


def _sc_scatter(h, g, srcf, dst3, dpt, t0, t1):
    npad, dp = h.shape
    sch = dst3.shape[2]
    wrow = g.shape[1]
    gpc = sch // 8
    rt = npad // NS

    def body(h_hbm, g_hbm, src_hbm, dst_hbm, acc_out,
             srcv, d0, d1, r0, r1, g0, g1, accsh,
             ga0, ga1, gb0, gb1, gd0, gd1, ss0, ss1):
        c = lax.axis_index("c")
        s = lax.axis_index("s")
        rows = (r0, r1)
        gvs = (g0, g1)
        dcs = (d0, d1)
        ga = (ga0, ga1)
        gb = (gb0, gb1)
        gd = (gd0, gd1)
        ss = (ss0, ss1)
        tilebase = jnp.where(c == 0, s * t0, NS * t0 + s * t1)
        tloc = jnp.where(c == 0, t0, t1)

        pltpu.sync_copy(src_hbm.at[pl.ds(tilebase * sch, t0 * sch)], srcv)

        z = jnp.zeros((16,), jnp.float32)
        for r in range(16):
            for q in range(dp // 16):
                r0[r, q * 16:(q + 1) * 16] = z

        def zl(j, carry):
            pltpu.sync_copy(r0.at[pl.ds(0, 16)],
                            accsh.at[pl.ds(s * rt + j * 16, 16)])
            return carry

        lax.fori_loop(0, rt // 16, zl, 0)
        plsc.subcore_barrier()

        def issue_gather(kk, cell):
            return pltpu.async_copy(h_hbm.at[srcv.at[pl.ds(kk * sch, sch)]],
                                    rows[cell], ga[cell])

        def wait_gather(cell):
            pltpu.make_async_copy(h_hbm.at[srcv.at[pl.ds(0, sch)]],
                                  rows[cell], ga[cell]).wait()

        def issue_g(kk, cell):
            base = (tilebase + kk) * gpc
            return pltpu.async_copy(g_hbm.at[pl.ds(base, gpc)], gvs[cell],
                                    gb[cell])

        def wait_g(cell):
            pltpu.make_async_copy(g_hbm.at[pl.ds(0, gpc)], gvs[cell],
                                  gb[cell]).wait()

        def issue_dst(kk, cell):
            return pltpu.async_copy(dst_hbm.at[tilebase + kk], dcs[cell],
                                    gd[cell])

        def wait_dst(cell):
            pltpu.make_async_copy(dst_hbm.at[0], dcs[cell], gd[cell]).wait()

        def issue_scatter(cell):
            return pltpu.async_copy(rows[cell], accsh.at[dcs[cell].at[0]],
                                    ss[cell], add=True)

        def wait_scatter(cell):
            pltpu.make_async_copy(rows[cell], accsh.at[dcs[cell].at[0]],
                                  ss[cell]).wait()

        def ew(cell):
            rr, gg = rows[cell], gvs[cell]

            def erow(j, carry2):
                for u in range(8):
                    r = j * 8 + u
                    for q in range(dpt // 16):
                        sl = pl.ds(q * 16, 16)
                        slg = pl.ds(u * dpt + q * 16, 16)
                        rr[r, sl] = jnp.maximum(rr[r, sl] + gg[j, slg], 0.0)
                return carry2

            lax.fori_loop(0, gpc, erow, 0)

        issue_gather(0, 0)
        issue_dst(0, 0)
        issue_g(0, 0)
        wait_gather(0)
        wait_g(0)
        issue_gather(1, 1)
        issue_dst(1, 1)
        issue_g(1, 1)
        ew(0)
        wait_dst(0)
        issue_scatter(0)
        wait_gather(1)
        wait_g(1)
        wait_scatter(0)
        issue_gather(2, 0)
        issue_dst(2, 0)
        issue_g(2, 0)
        ew(1)
        wait_dst(1)
        issue_scatter(1)

        def pair(j, carry):
            for u in range(2):
                k = 2 + 2 * j + u
                b = u
                wait_gather(b)
                wait_g(b)
                kk = jnp.minimum(k + 1, tloc - 1)
                wait_scatter(1 - b)
                issue_gather(kk, 1 - b)
                issue_dst(kk, 1 - b)
                issue_g(kk, 1 - b)
                ew(b)
                wait_dst(b)
                issue_scatter(b)
            return carry

        lax.fori_loop(0, (tloc - 2) // 2, pair, 0)

        wait_gather(0)
        wait_g(0)
        wait_dst(0)
        wait_scatter(1)
        plsc.subcore_barrier()

        def co(j, carry):
            off = s * rt + j * sch
            pltpu.sync_copy(accsh.at[pl.ds(off, sch)], r0)
            pltpu.sync_copy(r0, acc_out.at[c, pl.ds(off, sch)])
            return carry

        lax.fori_loop(0, rt // sch, co, 0)

    fn = pl.kernel(
        body,
        out_type=[jax.ShapeDtypeStruct((NC, npad, dp), jnp.float32)],
        mesh=_mesh(),
        scratch_types=[
            pltpu.VMEM((t0 * sch,), jnp.int32),
            pltpu.VMEM((1, sch), jnp.int32),
            pltpu.VMEM((1, sch), jnp.int32),
            pltpu.VMEM((sch, dp), jnp.float32),
            pltpu.VMEM((sch, dp), jnp.float32),
            pltpu.VMEM((gpc, wrow), jnp.float32),
            pltpu.VMEM((gpc, wrow), jnp.float32),
            pltpu.VMEM_SHARED((npad, dp), jnp.float32),
            pltpu.SemaphoreType.DMA,
            pltpu.SemaphoreType.DMA,
            pltpu.SemaphoreType.DMA,
            pltpu.SemaphoreType.DMA,
            pltpu.SemaphoreType.DMA,
            pltpu.SemaphoreType.DMA,
            pltpu.SemaphoreType.DMA,
            pltpu.SemaphoreType.DMA,
        ])
    (out,) = fn(h, g, srcf, dst3)
    return out


def _sc_degree(dstp, npad):
    ep = dstp.shape[0]
    t = ep // (NC * NS * CH)
    rt = npad // NS
    dp = 128

    def body(dst_hbm, cnt_out, dstv, onesv, cntsh):
        c = lax.axis_index("c")
        s = lax.axis_index("s")
        w = c * NS + s

        z = jnp.zeros((16,), jnp.float32)
        for r in range(16):
            for q in range(dp // 16):
                onesv[r, q * 16:(q + 1) * 16] = z

        def zl(j, carry):
            pltpu.sync_copy(onesv.at[pl.ds(0, 16)],
                            cntsh.at[pl.ds(s * rt + j * 16, 16)])
            return carry

        lax.fori_loop(0, rt // 16, zl, 0)

        one = jnp.ones((16,), jnp.float32)

        def ol(r, carry):
            for q in range(dp // 16):
                onesv[r, q * 16:(q + 1) * 16] = one
            return carry

        lax.fori_loop(0, CH, ol, 0)
        plsc.subcore_barrier()

        def chunk(k, carry):
            base = (w * t + k) * CH
            pltpu.sync_copy(dst_hbm.at[pl.ds(base, CH)], dstv)
            pltpu.sync_copy(onesv, cntsh.at[dstv], add=True)
            return carry

        lax.fori_loop(0, t, chunk, 0)
        plsc.subcore_barrier()

        def co(j, carry):
            off = s * rt + j * CH
            pltpu.sync_copy(cntsh.at[pl.ds(off, CH)], onesv)
            pltpu.sync_copy(onesv, cnt_out.at[c, pl.ds(off, CH)])
            return carry

        lax.fori_loop(0, rt // CH, co, 0)

    fn = pl.kernel(
        body,
        out_type=[jax.ShapeDtypeStruct((NC, npad, dp), jnp.float32)],
        mesh=_mesh(),
        scratch_types=[
            pltpu.VMEM((CH,), jnp.int32),
            pltpu.VMEM((CH, dp), jnp.float32),
            pltpu.VMEM_SHARED((npad, dp), jnp.float32),
        ])
    (out,) = fn(dstp)
    return out


def kernel(x, edge_index, edge_attr, params):
    n, d = x.shape
    e = edge_index.shape[1]
    de = edge_attr.shape[1]
    nlayers = len(params)

    npad = _ru(n + 1, 2048)
    ep = _ru(e, 2 * NW * CH)

    pad_dst = n + jnp.arange(ep - e, dtype=edge_index.dtype) % (npad - n)
    src = jnp.concatenate(
        [edge_index[0], jnp.zeros((ep - e,), edge_index.dtype)])
    dst = jnp.concatenate([edge_index[1], pad_dst])
    nchunks = ep // SCH
    t1 = (nchunks // NS) * 35 // 100 // 2 * 2
    t0 = nchunks // NS - t1
    srcf = jnp.concatenate(
        [src, jnp.zeros(((t0 - t1) * SCH,), src.dtype)])
    dst3 = dst.reshape(nchunks, 1, SCH)
    xp = jnp.pad(x, ((0, npad - n), (0, 0)))
    ea = jnp.pad(edge_attr, ((0, ep - e), (0, 0))).reshape(ep // 8, 8 * de)

    dps = [_ru(p["mW1"].shape[1], 128) for p in params]
    dpts = [_ru(p["mW1"].shape[1], 16) for p in params]
    douts = [p["mW1"].shape[1] for p in params]
    dins = [p["mW1"].shape[0] - de for p in params]
    i8 = jnp.eye(8, dtype=jnp.float32)

    lp0 = params[0]
    h = _node_linear(
        xp,
        _padw(lp0["mW1"][: dins[0]], d, dps[0]),
        _padw(lp0["mb1"][None, :], 1, dps[0]),
    )

    cntp = _sc_degree(dst, npad)
    for i, lp in enumerate(params):
        dp = dps[i]
        dpt = dpts[i]
        mw1e = _padw(lp["mW1"][dins[i]:], de, dp)
        ab1 = lp["ab1"][None, :]
        ab2 = lp["ab2"][None, :]
        ea, g = _attr_step(
            ea,
            jnp.kron(i8, lp["aW1"]), jnp.tile(ab1, (1, 8)),
            jnp.kron(i8, lp["aW2"]), jnp.tile(ab2, (1, 8)),
            jnp.kron(i8, _padw(lp["mW1"][dins[i]:], de, dpt)))
        p = _sc_scatter(h, g, srcf, dst3, dpt, t0, t1)
        last = i == nlayers - 1
        if last:
            wn = jnp.zeros((dp, dp), jnp.float32)
            bn = jnp.zeros((1, dp), jnp.float32)
        else:
            nxt = params[i + 1]
            wn = _padw(nxt["mW1"][: dins[i + 1]], dp, dps[i + 1])
            bn = _padw(nxt["mb1"][None, :], 1, dps[i + 1])
        h = _post_step(
            p, cntp, h, ab1, lp["aW2"], ab2, mw1e,
            _padw(lp["mW2"], dp, dp), _padw(lp["mb2"][None, :], 1, dp),
            wn, bn, last)

    return h[:n, : douts[-1]]


# speedup vs baseline: 1.0502x; 1.0210x over previous
import functools

import jax
import jax.numpy as jnp
from jax import lax
from jax.experimental import pallas as pl
from jax.experimental.pallas import tpu as pltpu
from jax.experimental.pallas import tpu_sc as plsc

NC = 2
NS = 16
NW = NC * NS
CH = 128
SCH = 64


def _ru(n, m):
    return (n + m - 1) // m * m


def _padw(w, r, c):
    return jnp.pad(w, ((0, r - w.shape[0]), (0, c - w.shape[1])))


def _linear_body(x_ref, w_ref, b_ref, o_ref):
    o_ref[...] = (
        jnp.dot(x_ref[...], w_ref[...], preferred_element_type=jnp.float32)
        + b_ref[...]
    )


def _node_linear(xp, w, b, blk=512):
    n, k = xp.shape
    dp = w.shape[1]
    return pl.pallas_call(
        _linear_body,
        grid=(n // blk,),
        in_specs=[
            pl.BlockSpec((blk, k), lambda i: (i, 0)),
            pl.BlockSpec((k, dp), lambda i: (0, 0)),
            pl.BlockSpec((1, dp), lambda i: (0, 0)),
        ],
        out_specs=pl.BlockSpec((blk, dp), lambda i: (i, 0)),
        out_shape=jax.ShapeDtypeStruct((n, dp), jnp.float32),
    )(xp, w, b)


def _attr_body(ea_ref, aw1_ref, ab1_ref, aw2_ref, ab2_ref, mw1e_ref,
               ean_ref, g_ref):
    h = jnp.maximum(
        jnp.dot(ea_ref[...], aw1_ref[...], preferred_element_type=jnp.float32)
        + ab1_ref[...], 0.0)
    ean = (jnp.dot(h, aw2_ref[...], preferred_element_type=jnp.float32)
           + ab2_ref[...])
    ean_ref[...] = ean
    g_ref[...] = jnp.dot(ean, mw1e_ref[...], preferred_element_type=jnp.float32)


def _attr_step(ea, kaw1, kab1, kaw2, kab2, kmw1e, blk=256):
    ep8, w16 = ea.shape
    wrow = kmw1e.shape[1]
    return pl.pallas_call(
        _attr_body,
        grid=(ep8 // blk,),
        in_specs=[
            pl.BlockSpec((blk, w16), lambda i: (i, 0)),
            pl.BlockSpec((w16, w16), lambda i: (0, 0)),
            pl.BlockSpec((1, w16), lambda i: (0, 0)),
            pl.BlockSpec((w16, w16), lambda i: (0, 0)),
            pl.BlockSpec((1, w16), lambda i: (0, 0)),
            pl.BlockSpec((w16, wrow), lambda i: (0, 0)),
        ],
        out_specs=[
            pl.BlockSpec((blk, w16), lambda i: (i, 0)),
            pl.BlockSpec((blk, wrow), lambda i: (i, 0)),
        ],
        out_shape=[
            jax.ShapeDtypeStruct((ep8, w16), jnp.float32),
            jax.ShapeDtypeStruct((ep8, wrow), jnp.float32),
        ],
    )(ea, kaw1, kab1, kaw2, kab2, kmw1e)


def _post_body(p_ref, c_ref, h_ref, ab1_ref, aw2_ref, ab2_ref, mw1e_ref,
               mw2_ref, mb2_ref, wn_ref, bn_ref, o_ref, *, last):
    gs = (jnp.dot(jnp.maximum(ab1_ref[...], 0.0), aw2_ref[...],
                  preferred_element_type=jnp.float32) + ab2_ref[...])
    gself = jnp.dot(gs, mw1e_ref[...], preferred_element_type=jnp.float32)
    acc = p_ref[0] + p_ref[1] + jnp.maximum(h_ref[...] + gself, 0.0)
    cnt = c_ref[0, :, 0:1] + c_ref[1, :, 0:1] + 1.0
    m = acc / cnt
    out = (jnp.dot(m, mw2_ref[...], preferred_element_type=jnp.float32)
           + mb2_ref[...])
    if last:
        o_ref[...] = out
    else:
        out = jnp.maximum(out, 0.0)
        o_ref[...] = (jnp.dot(out, wn_ref[...],
                              preferred_element_type=jnp.float32)
                      + bn_ref[...])


def _post_step(p, cntp, h, ab1, aw2, ab2, mw1e, mw2, mb2, wn, bn, last,
               blk=512):
    npad, dp = h.shape
    de = aw2.shape[0]
    dn = wn.shape[1]
    return pl.pallas_call(
        functools.partial(_post_body, last=last),
        grid=(npad // blk,),
        in_specs=[
            pl.BlockSpec((2, blk, dp), lambda i: (0, i, 0)),
            pl.BlockSpec((2, blk, 128), lambda i: (0, i, 0)),
            pl.BlockSpec((blk, dp), lambda i: (i, 0)),
            pl.BlockSpec((1, de), lambda i: (0, 0)),
            pl.BlockSpec((de, de), lambda i: (0, 0)),
            pl.BlockSpec((1, de), lambda i: (0, 0)),
            pl.BlockSpec((de, dp), lambda i: (0, 0)),
            pl.BlockSpec((dp, dp), lambda i: (0, 0)),
            pl.BlockSpec((1, dp), lambda i: (0, 0)),
            pl.BlockSpec((dp, dn), lambda i: (0, 0)),
            pl.BlockSpec((1, dn), lambda i: (0, 0)),
        ],
        out_specs=pl.BlockSpec((blk, dn), lambda i: (i, 0)),
        out_shape=jax.ShapeDtypeStruct((npad, dn), jnp.float32),
    )(p, cntp, h, ab1, aw2, ab2, mw1e, mw2, mb2, wn, bn)


def _mesh():
    return plsc.VectorSubcoreMesh(
        core_axis_name="c", subcore_axis_name="s", num_cores=NC,
        num_subcores=NS)


def _sc_scatter(h, g, srcf, dst3, dpt, t0, t1):
    npad, dp = h.shape
    sch = dst3.shape[2]
    wrow = g.shape[1]
    gpc = sch // 8
    rt = npad // NS

    def body(h_hbm, g_hbm, src_hbm, dst_hbm, acc_out,
             srcv, d0, d1, r0, r1, g0, g1, accsh,
             ga0, ga1, gb0, gb1, gd0, gd1, ss0, ss1):
        c = lax.axis_index("c")
        s = lax.axis_index("s")
        rows = (r0, r1)
        gvs = (g0, g1)
        dcs = (d0, d1)
        ga = (ga0, ga1)
        gb = (gb0, gb1)
        gd = (gd0, gd1)
        ss = (ss0, ss1)
        tilebase = jnp.where(c == 0, s * t0, NS * t0 + s * t1)
        tloc = jnp.where(c == 0, t0, t1)

        pltpu.sync_copy(src_hbm.at[pl.ds(tilebase * sch, t0 * sch)], srcv)

        z = jnp.zeros((16,), jnp.float32)
        for r in range(16):
            for q in range(dp // 16):
                r0[r, q * 16:(q + 1) * 16] = z

        def zl(j, carry):
            pltpu.sync_copy(r0.at[pl.ds(0, 16)],
                            accsh.at[pl.ds(s * rt + j * 16, 16)])
            return carry

        lax.fori_loop(0, rt // 16, zl, 0)
        plsc.subcore_barrier()

        def issue_gather(kk, cell):
            return pltpu.async_copy(h_hbm.at[srcv.at[pl.ds(kk * sch, sch)]],
                                    rows[cell], ga[cell])

        def wait_gather(cell):
            pltpu.make_async_copy(h_hbm.at[srcv.at[pl.ds(0, sch)]],
                                  rows[cell], ga[cell]).wait()

        def issue_g(kk, cell):
            base = (tilebase + kk) * gpc
            return pltpu.async_copy(g_hbm.at[pl.ds(base, gpc)], gvs[cell],
                                    gb[cell])

        def wait_g(cell):
            pltpu.make_async_copy(g_hbm.at[pl.ds(0, gpc)], gvs[cell],
                                  gb[cell]).wait()

        def issue_dst(kk, cell):
            return pltpu.async_copy(dst_hbm.at[tilebase + kk], dcs[cell],
                                    gd[cell])

        def wait_dst(cell):
            pltpu.make_async_copy(dst_hbm.at[0], dcs[cell], gd[cell]).wait()

        def issue_scatter(cell):
            return pltpu.async_copy(rows[cell], accsh.at[dcs[cell].at[0]],
                                    ss[cell], add=True)

        def wait_scatter(cell):
            pltpu.make_async_copy(rows[cell], accsh.at[dcs[cell].at[0]],
                                  ss[cell]).wait()

        def ew(cell):
            rr, gg = rows[cell], gvs[cell]

            def erow(j, carry2):
                for u in range(8):
                    r = j * 8 + u
                    for q in range(dpt // 16):
                        sl = pl.ds(q * 16, 16)
                        slg = pl.ds(u * dpt + q * 16, 16)
                        rr[r, sl] = jnp.maximum(rr[r, sl] + gg[j, slg], 0.0)
                return carry2

            lax.fori_loop(0, gpc, erow, 0)

        issue_gather(0, 0)
        issue_dst(0, 0)
        issue_g(0, 0)
        wait_gather(0)
        wait_g(0)
        issue_gather(1, 1)
        issue_dst(1, 1)
        issue_g(1, 1)
        ew(0)
        wait_dst(0)
        issue_scatter(0)
        wait_gather(1)
        wait_g(1)
        wait_scatter(0)
        issue_gather(2, 0)
        issue_dst(2, 0)
        issue_g(2, 0)
        ew(1)
        wait_dst(1)
        issue_scatter(1)

        def pair(j, carry):
            for u in range(2):
                k = 2 + 2 * j + u
                b = u
                wait_gather(b)
                wait_g(b)
                kk = jnp.minimum(k + 1, tloc - 1)
                wait_scatter(1 - b)
                issue_gather(kk, 1 - b)
                issue_dst(kk, 1 - b)
                issue_g(kk, 1 - b)
                ew(b)
                wait_dst(b)
                issue_scatter(b)
            return carry

        lax.fori_loop(0, (tloc - 2) // 2, pair, 0)

        wait_gather(0)
        wait_g(0)
        wait_dst(0)
        wait_scatter(1)
        plsc.subcore_barrier()

        def co(j, carry):
            off = s * rt + j * sch
            pltpu.sync_copy(accsh.at[pl.ds(off, sch)], r0)
            pltpu.sync_copy(r0, acc_out.at[c, pl.ds(off, sch)])
            return carry

        lax.fori_loop(0, rt // sch, co, 0)

    fn = pl.kernel(
        body,
        out_type=[jax.ShapeDtypeStruct((NC, npad, dp), jnp.float32)],
        mesh=_mesh(),
        scratch_types=[
            pltpu.VMEM((t0 * sch,), jnp.int32),
            pltpu.VMEM((1, sch), jnp.int32),
            pltpu.VMEM((1, sch), jnp.int32),
            pltpu.VMEM((sch, dp), jnp.float32),
            pltpu.VMEM((sch, dp), jnp.float32),
            pltpu.VMEM((gpc, wrow), jnp.float32),
            pltpu.VMEM((gpc, wrow), jnp.float32),
            pltpu.VMEM_SHARED((npad, dp), jnp.float32),
            pltpu.SemaphoreType.DMA,
            pltpu.SemaphoreType.DMA,
            pltpu.SemaphoreType.DMA,
            pltpu.SemaphoreType.DMA,
            pltpu.SemaphoreType.DMA,
            pltpu.SemaphoreType.DMA,
            pltpu.SemaphoreType.DMA,
            pltpu.SemaphoreType.DMA,
        ])
    (out,) = fn(h, g, srcf, dst3)
    return out


def _sc_degree(dstp, npad):
    ep = dstp.shape[0]
    t = ep // (NC * NS * CH)
    rt = npad // NS
    dp = 128

    def body(dst_hbm, cnt_out, dstv, onesv, cntsh):
        c = lax.axis_index("c")
        s = lax.axis_index("s")
        w = c * NS + s

        z = jnp.zeros((16,), jnp.float32)
        for r in range(16):
            for q in range(dp // 16):
                onesv[r, q * 16:(q + 1) * 16] = z

        def zl(j, carry):
            pltpu.sync_copy(onesv.at[pl.ds(0, 16)],
                            cntsh.at[pl.ds(s * rt + j * 16, 16)])
            return carry

        lax.fori_loop(0, rt // 16, zl, 0)

        one = jnp.ones((16,), jnp.float32)

        def ol(r, carry):
            for q in range(dp // 16):
                onesv[r, q * 16:(q + 1) * 16] = one
            return carry

        lax.fori_loop(0, CH, ol, 0)
        plsc.subcore_barrier()

        def chunk(k, carry):
            base = (w * t + k) * CH
            pltpu.sync_copy(dst_hbm.at[pl.ds(base, CH)], dstv)
            pltpu.sync_copy(onesv, cntsh.at[dstv], add=True)
            return carry

        lax.fori_loop(0, t, chunk, 0)
        plsc.subcore_barrier()

        def co(j, carry):
            off = s * rt + j * CH
            pltpu.sync_copy(cntsh.at[pl.ds(off, CH)], onesv)
            pltpu.sync_copy(onesv, cnt_out.at[c, pl.ds(off, CH)])
            return carry

        lax.fori_loop(0, rt // CH, co, 0)

    fn = pl.kernel(
        body,
        out_type=[jax.ShapeDtypeStruct((NC, npad, dp), jnp.float32)],
        mesh=_mesh(),
        scratch_types=[
            pltpu.VMEM((CH,), jnp.int32),
            pltpu.VMEM((CH, dp), jnp.float32),
            pltpu.VMEM_SHARED((npad, dp), jnp.float32),
        ])
    (out,) = fn(dstp)
    return out


def kernel(x, edge_index, edge_attr, params):
    n, d = x.shape
    e = edge_index.shape[1]
    de = edge_attr.shape[1]
    nlayers = len(params)

    npad = _ru(n + 1, 2048)
    ep = _ru(e, 2 * NW * CH)

    pad_dst = n + jnp.arange(ep - e, dtype=edge_index.dtype) % (npad - n)
    src = jnp.concatenate(
        [edge_index[0], jnp.zeros((ep - e,), edge_index.dtype)])
    dst = jnp.concatenate([edge_index[1], pad_dst])
    nchunks = ep // SCH
    tpt = nchunks // NS
    fracs = [0.35, 0.29, 0.26, 0.26]
    tsplit = []
    for f in fracs[: len(params)]:
        t1 = max(2, int(tpt * f) // 2 * 2)
        tsplit.append((tpt - t1, t1))
    over = max(17 * a + 15 * b for a, b in tsplit) - nchunks
    srcf = jnp.concatenate(
        [src, jnp.zeros((max(0, over) * SCH,), src.dtype)])
    dst3 = dst.reshape(nchunks, 1, SCH)
    xp = jnp.pad(x, ((0, npad - n), (0, 0)))
    ea = jnp.pad(edge_attr, ((0, ep - e), (0, 0))).reshape(ep // 8, 8 * de)

    dps = [_ru(p["mW1"].shape[1], 128) for p in params]
    dpts = [_ru(p["mW1"].shape[1], 16) for p in params]
    douts = [p["mW1"].shape[1] for p in params]
    dins = [p["mW1"].shape[0] - de for p in params]
    i8 = jnp.eye(8, dtype=jnp.float32)

    lp0 = params[0]
    h = _node_linear(
        xp,
        _padw(lp0["mW1"][: dins[0]], d, dps[0]),
        _padw(lp0["mb1"][None, :], 1, dps[0]),
    )

    cntp = _sc_degree(dst, npad)
    for i, lp in enumerate(params):
        dp = dps[i]
        dpt = dpts[i]
        mw1e = _padw(lp["mW1"][dins[i]:], de, dp)
        ab1 = lp["ab1"][None, :]
        ab2 = lp["ab2"][None, :]
        ea, g = _attr_step(
            ea,
            jnp.kron(i8, lp["aW1"]), jnp.tile(ab1, (1, 8)),
            jnp.kron(i8, lp["aW2"]), jnp.tile(ab2, (1, 8)),
            jnp.kron(i8, _padw(lp["mW1"][dins[i]:], de, dpt)))
        p = _sc_scatter(h, g, srcf, dst3, dpt, tsplit[i][0], tsplit[i][1])
        last = i == nlayers - 1
        if last:
            wn = jnp.zeros((dp, dp), jnp.float32)
            bn = jnp.zeros((1, dp), jnp.float32)
        else:
            nxt = params[i + 1]
            wn = _padw(nxt["mW1"][: dins[i + 1]], dp, dps[i + 1])
            bn = _padw(nxt["mb1"][None, :], 1, dps[i + 1])
        h = _post_step(
            p, cntp, h, ab1, lp["aW2"], ab2, mw1e,
            _padw(lp["mW2"], dp, dp), _padw(lp["mb2"][None, :], 1, dp),
            wn, bn, last)

    return h[:n, : douts[-1]]
